# baseline (speedup 1.0000x reference)
import functools

import jax
import jax.numpy as jnp
from jax import lax
from jax.experimental import pallas as pl
from jax.experimental.pallas import tpu as pltpu
from jax.experimental.pallas import tpu_sc as plsc

_TILE = 512
_K = 8

_DN_T = (((1,), (1,)), ((), ()))


def _tc1_body(pr_ref, pa_ref, fr_ref, W1_ref, b1_ref, W2_ref, b2_ref,
              bp_ref, idx_ref):
    N = pa_ref.shape[1]
    p_row = pr_ref[0]
    p_all = pa_ref[0]
    f_r = fr_ref[0]

    pp = jax.lax.dot_general(p_row, p_all, _DN_T,
                             preferred_element_type=jnp.float32)
    pn_row = jnp.sum(p_row * p_row, axis=1, keepdims=True)
    pn_all = jnp.sum(p_all * p_all, axis=1, keepdims=True).T
    d2 = (pn_row + pn_all) - 2.0 * pp

    col = lax.broadcasted_iota(jnp.int32, d2.shape, 1)
    keys = (lax.bitcast_convert_type(d2, jnp.int32) & ~jnp.int32(2047)) | col

    big = jnp.int32(0x7FFFFFFF)
    work = keys
    ms = []
    for k in range(_K):
        m = jnp.min(work, axis=1, keepdims=True)
        ms.append(m)
        if k < _K - 1:
            work = jnp.where(work <= m, big, work)
    b = pl.program_id(0)
    idx8 = (jnp.concatenate(ms, axis=1) & jnp.int32(2047)) + b * N

    h = jnp.maximum(
        jax.lax.dot(f_r, W1_ref[...], preferred_element_type=jnp.float32,
                    precision=jax.lax.Precision.HIGHEST) + b1_ref[...], 0.0)
    logits = jax.lax.dot(h, W2_ref[...], preferred_element_type=jnp.float32,
                         precision=jax.lax.Precision.HIGHEST) + b2_ref[...]
    bp_ref[0] = jax.nn.sigmoid(logits)
    idx_ref[0] = idx8


def _make_sc_fd2(BN, D, NC, NS, L):
    NW = NC * NS
    per_w = BN // NW
    CH = 16
    chunks = per_w // CH

    mesh = plsc.VectorSubcoreMesh(core_axis_name="c", subcore_axis_name="s")

    @functools.partial(
        pl.kernel, mesh=mesh,
        out_type=jax.ShapeDtypeStruct((BN * _K, L), jnp.float32),
        scratch_types=[
            pltpu.VMEM((CH * _K,), jnp.int32),
            pltpu.VMEM((CH * _K, D), jnp.float32),
            pltpu.VMEM((CH, D), jnp.float32),
            pltpu.VMEM((CH * _K, L), jnp.float32),
            pltpu.SemaphoreType.DMA,
        ],
    )
    def sc_fd2(f_hbm, idx_hbm, out_hbm, idx_v, rows_v, cent_v, stage_v, sem):
        wid = lax.axis_index("s") * NC + lax.axis_index("c")
        base_c = wid * per_w

        def chunk_body(ci, _):
            cbase = base_c + ci * CH
            pbase = cbase * _K
            pltpu.sync_copy(idx_hbm.at[pl.ds(pbase, CH * _K)], idx_v)
            pltpu.async_copy(f_hbm.at[idx_v], rows_v, sem).wait()
            pltpu.sync_copy(f_hbm.at[pl.ds(cbase, CH)], cent_v)

            def centre_body(i, _):
                cvs = [cent_v[i, pl.ds(16 * j, 16)] for j in range(D // 16)]
                for kk in range(_K):
                    p = i * _K + kk
                    acc = jnp.zeros((L,), jnp.float32)
                    for j in range(D // 16):
                        dd = rows_v[p, pl.ds(16 * j, 16)] - cvs[j]
                        acc = acc + dd * dd
                    stage_v[p] = acc
                return 0

            lax.fori_loop(0, CH, centre_body, 0)
            pltpu.sync_copy(stage_v, out_hbm.at[pl.ds(pbase, CH * _K)])
            return 0

        lax.fori_loop(0, chunks, chunk_body, 0)

    return sc_fd2


def _tc2_body(fr_ref, bp_ref, fdp_ref, enh_ref):
    f_r = fr_ref[0]
    bp = bp_ref[0]
    fdp = fdp_ref[0]
    gsel = (lax.broadcasted_iota(jnp.int32, (_K * 16, _K), 0) // 16
            == lax.broadcasted_iota(jnp.int32, (_K * 16, _K), 1))
    fd2 = jax.lax.dot(fdp, gsel.astype(jnp.float32),
                      preferred_element_type=jnp.float32)
    fd = jnp.sqrt(fd2)
    fg = jnp.sum(fd, axis=1, keepdims=True) * (1.0 / _K)
    enh_ref[0] = f_r + 0.3 * (jnp.tanh(5.0 * fg) * bp)


@functools.partial(jax.jit, static_argnames=("interpret",))
def kernel(features, points, W1, b1, W2, b2, interpret=False):
    B, N, D = features.shape
    BN = B * N
    b1r = b1.reshape(1, -1)
    W2r = W2.reshape(-1, 1)
    b2r = b2.reshape(1, 1)

    grid = (B, N // _TILE)
    bp, idx = pl.pallas_call(
        _tc1_body,
        grid=grid,
        in_specs=[
            pl.BlockSpec((1, _TILE, 3), lambda b, t: (b, t, 0)),
            pl.BlockSpec((1, N, 3), lambda b, t: (b, 0, 0)),
            pl.BlockSpec((1, _TILE, D), lambda b, t: (b, t, 0)),
            pl.BlockSpec((D, 64), lambda b, t: (0, 0)),
            pl.BlockSpec((1, 64), lambda b, t: (0, 0)),
            pl.BlockSpec((64, 1), lambda b, t: (0, 0)),
            pl.BlockSpec((1, 1), lambda b, t: (0, 0)),
        ],
        out_specs=[
            pl.BlockSpec((1, _TILE, 1), lambda b, t: (b, t, 0)),
            pl.BlockSpec((1, _TILE, _K), lambda b, t: (b, t, 0)),
        ],
        out_shape=[
            jax.ShapeDtypeStruct((B, N, 1), jnp.float32),
            jax.ShapeDtypeStruct((B, N, _K), jnp.int32),
        ],
        interpret=interpret,
    )(points, points, features, W1, b1r, W2r, b2r)

    f_flat = features.reshape(BN, D)
    idx_flat = idx.reshape(BN * _K)
    fdp_flat = _make_sc_fd2(BN, D, 2, 16, 16)(f_flat, idx_flat)
    fdp = fdp_flat.reshape(B, N, _K * 16)

    enh = pl.pallas_call(
        _tc2_body,
        grid=grid,
        in_specs=[
            pl.BlockSpec((1, _TILE, D), lambda b, t: (b, t, 0)),
            pl.BlockSpec((1, _TILE, 1), lambda b, t: (b, t, 0)),
            pl.BlockSpec((1, _TILE, _K * 16), lambda b, t: (b, t, 0)),
        ],
        out_specs=pl.BlockSpec((1, _TILE, D), lambda b, t: (b, t, 0)),
        out_shape=jax.ShapeDtypeStruct((B, N, D), jnp.float32),
        interpret=interpret,
    )(features, bp, fdp)
    return (bp, enh)


# speedup vs baseline: 1.2695x; 1.2695x over previous
import functools

import jax
import jax.numpy as jnp
from jax import lax
from jax.experimental import pallas as pl
from jax.experimental.pallas import tpu as pltpu
from jax.experimental.pallas import tpu_sc as plsc

_TILE = 512
_K = 8

_DN_T = (((1,), (1,)), ((), ()))


def _tc1_body(pr_ref, pa_ref, fr_ref, W1_ref, b1_ref, W2_ref, b2_ref,
              bp_ref, idx_ref):
    N = pa_ref.shape[1]
    p_row = pr_ref[0]
    p_all = pa_ref[0]
    f_r = fr_ref[0]

    pp = jax.lax.dot_general(p_row, p_all, _DN_T,
                             preferred_element_type=jnp.float32)
    pn_row = jnp.sum(p_row * p_row, axis=1, keepdims=True)
    pn_all = jnp.sum(p_all * p_all, axis=1, keepdims=True).T
    d2 = (pn_row + pn_all) - 2.0 * pp

    col = lax.broadcasted_iota(jnp.int32, d2.shape, 1)
    keys = (lax.bitcast_convert_type(d2, jnp.int32) & ~jnp.int32(2047)) | col

    big = jnp.int32(0x7FFFFFFF)
    work = keys
    ms = []
    for k in range(_K):
        m = jnp.min(work, axis=1, keepdims=True)
        ms.append(m)
        if k < _K - 1:
            work = jnp.where(work <= m, big, work)
    b = pl.program_id(0)
    idx8 = (jnp.concatenate(ms, axis=1) & jnp.int32(2047)) + b * N

    h = jnp.maximum(
        jax.lax.dot(f_r, W1_ref[...], preferred_element_type=jnp.float32,
                    precision=jax.lax.Precision.HIGHEST) + b1_ref[...], 0.0)
    logits = jax.lax.dot(h, W2_ref[...], preferred_element_type=jnp.float32,
                         precision=jax.lax.Precision.HIGHEST) + b2_ref[...]
    bp_ref[0] = jax.nn.sigmoid(logits)
    idx_ref[0] = idx8


def _make_sc_fd2(BN, D, NC, NS, L):
    NW = NC * NS
    per_w = BN // NW
    CH = 16
    chunks = per_w // CH

    mesh = plsc.VectorSubcoreMesh(core_axis_name="c", subcore_axis_name="s")
    NACC = 4

    @functools.partial(
        pl.kernel, mesh=mesh,
        out_type=jax.ShapeDtypeStruct((BN * _K, L), jnp.float32),
        scratch_types=[
            pltpu.VMEM((per_w * _K,), jnp.int32),
            pltpu.VMEM((2, CH * _K, D), jnp.float32),
            pltpu.VMEM((2, CH, D), jnp.float32),
            pltpu.VMEM((CH * _K, L), jnp.float32),
            pltpu.SemaphoreType.DMA,
            pltpu.SemaphoreType.DMA,
            pltpu.SemaphoreType.DMA,
            pltpu.SemaphoreType.DMA,
        ],
    )
    def sc_fd2(f_hbm, idx_hbm, out_hbm, idx_all, rows_v, cent_v, stage_v,
               semr0, semr1, semc0, semc1):
        wid = lax.axis_index("s") * NC + lax.axis_index("c")
        base_c = wid * per_w
        semr = (semr0, semr1)
        semc = (semc0, semc1)

        pltpu.sync_copy(idx_hbm.at[pl.ds(base_c * _K, per_w * _K)], idx_all)

        def start_chunk(ci, buf):
            pltpu.async_copy(f_hbm.at[idx_all.at[pl.ds(ci * CH * _K, CH * _K)]],
                             rows_v.at[buf], semr[buf])
            pltpu.async_copy(f_hbm.at[pl.ds(base_c + ci * CH, CH)],
                             cent_v.at[buf], semc[buf])

        def wait_chunk(ci, buf):
            pltpu.make_async_copy(
                f_hbm.at[pl.ds(0, CH * _K)], rows_v.at[buf], semr[buf]).wait()
            pltpu.make_async_copy(
                f_hbm.at[pl.ds(0, CH)], cent_v.at[buf], semc[buf]).wait()

        def compute_chunk(ci, buf):
            def centre_body(i, _):
                cvs = [cent_v[buf, i, pl.ds(16 * j, 16)]
                       for j in range(D // 16)]
                for kk in range(_K):
                    p = i * _K + kk
                    accs = [jnp.zeros((L,), jnp.float32) for _ in range(NACC)]
                    for j in range(D // 16):
                        dd = rows_v[buf, p, pl.ds(16 * j, 16)] - cvs[j]
                        accs[j % NACC] = accs[j % NACC] + dd * dd
                    acc = (accs[0] + accs[1]) + (accs[2] + accs[3])
                    stage_v[p] = acc
                return 0

            lax.fori_loop(0, CH, centre_body, 0)
            pltpu.sync_copy(stage_v,
                            out_hbm.at[pl.ds((base_c + ci * CH) * _K,
                                             CH * _K)])

        start_chunk(0, 0)

        def super_body(s, _):
            for par in range(2):
                ci = 2 * s + par
                nxt = ci + 1

                @pl.when(nxt < chunks)
                def _():
                    start_chunk(nxt, 1 - par)

                wait_chunk(ci, par)
                compute_chunk(ci, par)
            return 0

        lax.fori_loop(0, chunks // 2, super_body, 0)

    return sc_fd2


def _tc2_body(fr_ref, bp_ref, fdp_ref, enh_ref):
    f_r = fr_ref[0]
    bp = bp_ref[0]
    fdp = fdp_ref[0]
    gsel = (lax.broadcasted_iota(jnp.int32, (_K * 16, _K), 0) // 16
            == lax.broadcasted_iota(jnp.int32, (_K * 16, _K), 1))
    fd2 = jax.lax.dot(fdp, gsel.astype(jnp.float32),
                      preferred_element_type=jnp.float32)
    fd = jnp.sqrt(fd2)
    fg = jnp.sum(fd, axis=1, keepdims=True) * (1.0 / _K)
    enh_ref[0] = f_r + 0.3 * (jnp.tanh(5.0 * fg) * bp)


@functools.partial(jax.jit, static_argnames=("interpret",))
def kernel(features, points, W1, b1, W2, b2, interpret=False):
    B, N, D = features.shape
    BN = B * N
    b1r = b1.reshape(1, -1)
    W2r = W2.reshape(-1, 1)
    b2r = b2.reshape(1, 1)

    grid = (B, N // _TILE)
    bp, idx = pl.pallas_call(
        _tc1_body,
        grid=grid,
        in_specs=[
            pl.BlockSpec((1, _TILE, 3), lambda b, t: (b, t, 0)),
            pl.BlockSpec((1, N, 3), lambda b, t: (b, 0, 0)),
            pl.BlockSpec((1, _TILE, D), lambda b, t: (b, t, 0)),
            pl.BlockSpec((D, 64), lambda b, t: (0, 0)),
            pl.BlockSpec((1, 64), lambda b, t: (0, 0)),
            pl.BlockSpec((64, 1), lambda b, t: (0, 0)),
            pl.BlockSpec((1, 1), lambda b, t: (0, 0)),
        ],
        out_specs=[
            pl.BlockSpec((1, _TILE, 1), lambda b, t: (b, t, 0)),
            pl.BlockSpec((1, _TILE, _K), lambda b, t: (b, t, 0)),
        ],
        out_shape=[
            jax.ShapeDtypeStruct((B, N, 1), jnp.float32),
            jax.ShapeDtypeStruct((B, N, _K), jnp.int32),
        ],
        interpret=interpret,
    )(points, points, features, W1, b1r, W2r, b2r)

    f_flat = features.reshape(BN, D)
    idx_flat = idx.reshape(BN * _K)
    fdp_flat = _make_sc_fd2(BN, D, 2, 16, 16)(f_flat, idx_flat)
    fdp = fdp_flat.reshape(B, N, _K * 16)

    enh = pl.pallas_call(
        _tc2_body,
        grid=grid,
        in_specs=[
            pl.BlockSpec((1, _TILE, D), lambda b, t: (b, t, 0)),
            pl.BlockSpec((1, _TILE, 1), lambda b, t: (b, t, 0)),
            pl.BlockSpec((1, _TILE, _K * 16), lambda b, t: (b, t, 0)),
        ],
        out_specs=pl.BlockSpec((1, _TILE, D), lambda b, t: (b, t, 0)),
        out_shape=jax.ShapeDtypeStruct((B, N, D), jnp.float32),
        interpret=interpret,
    )(features, bp, fdp)
    return (bp, enh)


# speedup vs baseline: 1.3289x; 1.0468x over previous
import functools

import jax
import jax.numpy as jnp
from jax import lax
from jax.experimental import pallas as pl
from jax.experimental.pallas import tpu as pltpu
from jax.experimental.pallas import tpu_sc as plsc

_TILE = 512
_K = 8

_DN_T = (((1,), (1,)), ((), ()))


def _tc1_body(pr_ref, pa_ref, fr_ref, W1_ref, b1_ref, W2_ref, b2_ref,
              bp_ref, idx_ref):
    N = pa_ref.shape[1]
    p_row = pr_ref[0]
    p_all = pa_ref[0]
    f_r = fr_ref[0]

    pp = jax.lax.dot_general(p_row, p_all, _DN_T,
                             preferred_element_type=jnp.float32)
    pn_row = jnp.sum(p_row * p_row, axis=1, keepdims=True)
    pn_all = jnp.sum(p_all * p_all, axis=1, keepdims=True).T
    d2 = (pn_row + pn_all) - 2.0 * pp

    col = lax.broadcasted_iota(jnp.int32, d2.shape, 1)
    keys = (lax.bitcast_convert_type(d2, jnp.int32) & ~jnp.int32(2047)) | col

    big = jnp.int32(0x7FFFFFFF)
    work = keys
    ms = []
    for k in range(_K):
        m = jnp.min(work, axis=1, keepdims=True)
        ms.append(m)
        if k < _K - 1:
            work = jnp.where(work <= m, big, work)
    b = pl.program_id(0)
    idx8 = (jnp.concatenate(ms, axis=1) & jnp.int32(2047)) + b * N

    h = jnp.maximum(
        jax.lax.dot(f_r, W1_ref[...], preferred_element_type=jnp.float32,
                    precision=jax.lax.Precision.HIGHEST) + b1_ref[...], 0.0)
    logits = jax.lax.dot(h, W2_ref[...], preferred_element_type=jnp.float32,
                         precision=jax.lax.Precision.HIGHEST) + b2_ref[...]
    bp_ref[0] = jax.nn.sigmoid(logits)
    idx_ref[0] = idx8


def _make_sc_fd2(BN, D, NC, NS, L):
    NW = NC * NS
    per_w = BN // NW
    CH = 16
    chunks = per_w // CH

    mesh = plsc.VectorSubcoreMesh(core_axis_name="c", subcore_axis_name="s")
    NACC = 8

    @functools.partial(
        pl.kernel, mesh=mesh,
        out_type=jax.ShapeDtypeStruct((BN * _K, L), jnp.float32),
        scratch_types=[
            pltpu.VMEM((per_w * _K,), jnp.int32),
            pltpu.VMEM((2, CH * _K, D), jnp.float32),
            pltpu.VMEM((2, CH, D), jnp.float32),
            pltpu.VMEM((2, CH * _K, L), jnp.float32),
            pltpu.SemaphoreType.DMA,
            pltpu.SemaphoreType.DMA,
            pltpu.SemaphoreType.DMA,
            pltpu.SemaphoreType.DMA,
            pltpu.SemaphoreType.DMA,
            pltpu.SemaphoreType.DMA,
        ],
    )
    def sc_fd2(f_hbm, idx_hbm, out_hbm, idx_all, rows_v, cent_v, stage_v,
               semr0, semr1, semc0, semc1, semo0, semo1):
        wid = lax.axis_index("s") * NC + lax.axis_index("c")
        base_c = wid * per_w
        semr = (semr0, semr1)
        semc = (semc0, semc1)
        semo = (semo0, semo1)

        pltpu.sync_copy(idx_hbm.at[pl.ds(base_c * _K, per_w * _K)], idx_all)

        def start_chunk(ci, buf):
            pltpu.async_copy(f_hbm.at[idx_all.at[pl.ds(ci * CH * _K, CH * _K)]],
                             rows_v.at[buf], semr[buf])
            pltpu.async_copy(f_hbm.at[pl.ds(base_c + ci * CH, CH)],
                             cent_v.at[buf], semc[buf])

        def wait_chunk(ci, buf):
            pltpu.make_async_copy(
                f_hbm.at[pl.ds(0, CH * _K)], rows_v.at[buf], semr[buf]).wait()
            pltpu.make_async_copy(
                f_hbm.at[pl.ds(0, CH)], cent_v.at[buf], semc[buf]).wait()

        def compute_chunk(ci, buf):
            def centre_body(i, _):
                cvs = [cent_v[buf, i, pl.ds(16 * j, 16)]
                       for j in range(D // 16)]
                for kk in range(_K):
                    p = i * _K + kk
                    accs = [None] * NACC
                    for j in range(D // 16):
                        dd = rows_v[buf, p, pl.ds(16 * j, 16)] - cvs[j]
                        sq = dd * dd
                        accs[j % NACC] = (sq if accs[j % NACC] is None
                                          else accs[j % NACC] + sq)
                    while len(accs) > 1:
                        accs = [accs[2 * a] + accs[2 * a + 1]
                                for a in range(len(accs) // 2)]
                    stage_v[buf, p] = accs[0]
                return 0

            lax.fori_loop(0, CH, centre_body, 0)
            pltpu.async_copy(stage_v.at[buf],
                             out_hbm.at[pl.ds((base_c + ci * CH) * _K,
                                              CH * _K)], semo[buf])

        def wait_out(buf):
            pltpu.make_async_copy(
                stage_v.at[buf], out_hbm.at[pl.ds(0, CH * _K)],
                semo[buf]).wait()

        start_chunk(0, 0)

        def super_body(s, _):
            for par in range(2):
                ci = 2 * s + par
                nxt = ci + 1

                @pl.when(nxt < chunks)
                def _():
                    start_chunk(nxt, 1 - par)

                wait_chunk(ci, par)

                @pl.when(ci >= 2)
                def _():
                    wait_out(par)

                compute_chunk(ci, par)
            return 0

        lax.fori_loop(0, chunks // 2, super_body, 0)
        wait_out(0)
        wait_out(1)

    return sc_fd2


def _tc2_body(fr_ref, bp_ref, fdp_ref, enh_ref):
    f_r = fr_ref[0]
    bp = bp_ref[0]
    fdp = fdp_ref[0]
    gsel = (lax.broadcasted_iota(jnp.int32, (_K * 16, _K), 0) // 16
            == lax.broadcasted_iota(jnp.int32, (_K * 16, _K), 1))
    fd2 = jax.lax.dot(fdp, gsel.astype(jnp.float32),
                      preferred_element_type=jnp.float32)
    fd = jnp.sqrt(fd2)
    fg = jnp.sum(fd, axis=1, keepdims=True) * (1.0 / _K)
    enh_ref[0] = f_r + 0.3 * (jnp.tanh(5.0 * fg) * bp)


@functools.partial(jax.jit, static_argnames=("interpret",))
def kernel(features, points, W1, b1, W2, b2, interpret=False):
    B, N, D = features.shape
    BN = B * N
    b1r = b1.reshape(1, -1)
    W2r = W2.reshape(-1, 1)
    b2r = b2.reshape(1, 1)

    grid = (B, N // _TILE)
    bp, idx = pl.pallas_call(
        _tc1_body,
        grid=grid,
        in_specs=[
            pl.BlockSpec((1, _TILE, 3), lambda b, t: (b, t, 0)),
            pl.BlockSpec((1, N, 3), lambda b, t: (b, 0, 0)),
            pl.BlockSpec((1, _TILE, D), lambda b, t: (b, t, 0)),
            pl.BlockSpec((D, 64), lambda b, t: (0, 0)),
            pl.BlockSpec((1, 64), lambda b, t: (0, 0)),
            pl.BlockSpec((64, 1), lambda b, t: (0, 0)),
            pl.BlockSpec((1, 1), lambda b, t: (0, 0)),
        ],
        out_specs=[
            pl.BlockSpec((1, _TILE, 1), lambda b, t: (b, t, 0)),
            pl.BlockSpec((1, _TILE, _K), lambda b, t: (b, t, 0)),
        ],
        out_shape=[
            jax.ShapeDtypeStruct((B, N, 1), jnp.float32),
            jax.ShapeDtypeStruct((B, N, _K), jnp.int32),
        ],
        interpret=interpret,
    )(points, points, features, W1, b1r, W2r, b2r)

    f_flat = features.reshape(BN, D)
    idx_flat = idx.reshape(BN * _K)
    fdp_flat = _make_sc_fd2(BN, D, 2, 16, 16)(f_flat, idx_flat)
    fdp = fdp_flat.reshape(B, N, _K * 16)

    enh = pl.pallas_call(
        _tc2_body,
        grid=grid,
        in_specs=[
            pl.BlockSpec((1, _TILE, D), lambda b, t: (b, t, 0)),
            pl.BlockSpec((1, _TILE, 1), lambda b, t: (b, t, 0)),
            pl.BlockSpec((1, _TILE, _K * 16), lambda b, t: (b, t, 0)),
        ],
        out_specs=pl.BlockSpec((1, _TILE, D), lambda b, t: (b, t, 0)),
        out_shape=jax.ShapeDtypeStruct((B, N, D), jnp.float32),
        interpret=interpret,
    )(features, bp, fdp)
    return (bp, enh)


# speedup vs baseline: 2.2063x; 1.6603x over previous
import functools

import jax
import jax.numpy as jnp
from jax.experimental import pallas as pl

_TILE = 1024
_K = 8

_DN_T = (((1,), (1,)), ((), ()))


def _body(pr_ref, pa_ref, fr_ref, fa_ref, W1_ref, b1_ref, W2_ref, b2_ref,
          bp_ref, enh_ref):
    p_row = pr_ref[0]
    p_all = pa_ref[0]
    f_r = fr_ref[0]
    f_a = fa_ref[0]

    pp = jax.lax.dot_general(p_row, p_all, _DN_T,
                             preferred_element_type=jnp.float32)
    pn_row = jnp.sum(p_row * p_row, axis=1, keepdims=True)
    pn_all = jnp.sum(p_all * p_all, axis=1, keepdims=True).T
    d2 = (pn_row + pn_all) - 2.0 * pp

    big = jnp.float32(3e38)
    work = d2
    m = None
    for k in range(_K):
        m = jnp.min(work, axis=1, keepdims=True)
        if k < _K - 1:
            work = jnp.where(work <= m, big, work)
    mask = d2 <= m

    fn_all = jnp.sum(f_a * f_a, axis=1, keepdims=True).T
    fn_row = jnp.sum(f_r * f_r, axis=1, keepdims=True)
    gram = jax.lax.dot_general(f_r, f_a, _DN_T,
                               preferred_element_type=jnp.float32)
    fd2 = jnp.maximum(fn_row + (fn_all - 2.0 * gram), 0.0)
    fd = jnp.sqrt(fd2)
    acc = jnp.sum(jnp.where(mask, fd, 0.0), axis=1, keepdims=True)
    fg = acc * (1.0 / _K)

    h = jnp.maximum(
        jax.lax.dot(f_r, W1_ref[...], preferred_element_type=jnp.float32,
                    precision=jax.lax.Precision.HIGHEST) + b1_ref[...], 0.0)
    logits = jax.lax.dot(h, W2_ref[...], preferred_element_type=jnp.float32,
                         precision=jax.lax.Precision.HIGHEST) + b2_ref[...]
    bp = jax.nn.sigmoid(logits)

    enh = f_r + 0.3 * (jnp.tanh(5.0 * fg) * bp)
    bp_ref[0] = bp
    enh_ref[0] = enh


@functools.partial(jax.jit, static_argnames=("interpret",))
def kernel(features, points, W1, b1, W2, b2, interpret=False):
    B, N, D = features.shape
    b1r = b1.reshape(1, -1)
    W2r = W2.reshape(-1, 1)
    b2r = b2.reshape(1, 1)

    grid = (B, N // _TILE)
    bp, enh = pl.pallas_call(
        _body,
        grid=grid,
        in_specs=[
            pl.BlockSpec((1, _TILE, 3), lambda b, t: (b, t, 0)),
            pl.BlockSpec((1, N, 3), lambda b, t: (b, 0, 0)),
            pl.BlockSpec((1, _TILE, D), lambda b, t: (b, t, 0)),
            pl.BlockSpec((1, N, D), lambda b, t: (b, 0, 0)),
            pl.BlockSpec((D, 64), lambda b, t: (0, 0)),
            pl.BlockSpec((1, 64), lambda b, t: (0, 0)),
            pl.BlockSpec((64, 1), lambda b, t: (0, 0)),
            pl.BlockSpec((1, 1), lambda b, t: (0, 0)),
        ],
        out_specs=[
            pl.BlockSpec((1, _TILE, 1), lambda b, t: (b, t, 0)),
            pl.BlockSpec((1, _TILE, D), lambda b, t: (b, t, 0)),
        ],
        out_shape=[
            jax.ShapeDtypeStruct((B, N, 1), jnp.float32),
            jax.ShapeDtypeStruct((B, N, D), jnp.float32),
        ],
        interpret=interpret,
    )(points, points, features, features, W1, b1r, W2r, b2r)
    return (bp, enh)


# speedup vs baseline: 2.4049x; 1.0900x over previous
import functools

import jax
import jax.numpy as jnp
from jax.experimental import pallas as pl

_TILE = 1024
_K = 8

_DN_T = (((1,), (1,)), ((), ()))


def _body(pr_ref, pa_ref, fr_ref, fa_ref, W1_ref, b1_ref, W2_ref, b2_ref,
          bp_ref, enh_ref):
    p_row = pr_ref[0]
    p_all = pa_ref[0]
    f_r = fr_ref[0]
    f_a = fa_ref[0]

    N = pa_ref.shape[1]
    D = fa_ref.shape[2]
    one3 = jnp.ones((1, 3), jnp.float32)
    oneD = jnp.ones((1, D), jnp.float32)

    pp = jax.lax.dot_general(p_row, p_all, _DN_T,
                             preferred_element_type=jnp.float32)
    pn_row = jnp.sum(p_row * p_row, axis=1, keepdims=True)
    pn_all = jax.lax.dot_general(one3, p_all * p_all, _DN_T,
                                 preferred_element_type=jnp.float32)
    d2 = (pn_row + pn_all) - 2.0 * pp

    big = jnp.float32(3e38)
    work = d2
    m = None
    for k in range(_K):
        m = jnp.min(work, axis=1, keepdims=True)
        if k < _K - 1:
            work = jnp.where(work <= m, big, work)
    mask = d2 <= m

    fn_all = jax.lax.dot_general(oneD, f_a * f_a, _DN_T,
                                 preferred_element_type=jnp.float32)
    fn_row = jnp.sum(f_r * f_r, axis=1, keepdims=True)
    gram = jax.lax.dot_general(f_r, f_a, _DN_T,
                               preferred_element_type=jnp.float32)
    fd2 = jnp.maximum(fn_row + (fn_all - 2.0 * gram), 0.0)
    fd = jnp.sqrt(fd2)
    oneN = jnp.ones((N, 1), jnp.float32)
    acc = jax.lax.dot(jnp.where(mask, fd, 0.0), oneN,
                      preferred_element_type=jnp.float32)
    fg = acc * (1.0 / _K)

    h = jnp.maximum(
        jax.lax.dot(f_r, W1_ref[...], preferred_element_type=jnp.float32,
                    precision=jax.lax.Precision.HIGHEST) + b1_ref[...], 0.0)
    logits = jax.lax.dot(h, W2_ref[...], preferred_element_type=jnp.float32,
                         precision=jax.lax.Precision.HIGHEST) + b2_ref[...]
    bp = jax.nn.sigmoid(logits)

    enh = f_r + 0.3 * (jnp.tanh(5.0 * fg) * bp)
    bp_ref[0] = bp
    enh_ref[0] = enh


@functools.partial(jax.jit, static_argnames=("interpret",))
def kernel(features, points, W1, b1, W2, b2, interpret=False):
    B, N, D = features.shape
    b1r = b1.reshape(1, -1)
    W2r = W2.reshape(-1, 1)
    b2r = b2.reshape(1, 1)

    grid = (B, N // _TILE)
    bp, enh = pl.pallas_call(
        _body,
        grid=grid,
        in_specs=[
            pl.BlockSpec((1, _TILE, 3), lambda b, t: (b, t, 0)),
            pl.BlockSpec((1, N, 3), lambda b, t: (b, 0, 0)),
            pl.BlockSpec((1, _TILE, D), lambda b, t: (b, t, 0)),
            pl.BlockSpec((1, N, D), lambda b, t: (b, 0, 0)),
            pl.BlockSpec((D, 64), lambda b, t: (0, 0)),
            pl.BlockSpec((1, 64), lambda b, t: (0, 0)),
            pl.BlockSpec((64, 1), lambda b, t: (0, 0)),
            pl.BlockSpec((1, 1), lambda b, t: (0, 0)),
        ],
        out_specs=[
            pl.BlockSpec((1, _TILE, 1), lambda b, t: (b, t, 0)),
            pl.BlockSpec((1, _TILE, D), lambda b, t: (b, t, 0)),
        ],
        out_shape=[
            jax.ShapeDtypeStruct((B, N, 1), jnp.float32),
            jax.ShapeDtypeStruct((B, N, D), jnp.float32),
        ],
        interpret=interpret,
    )(points, points, features, features, W1, b1r, W2r, b2r)
    return (bp, enh)
